# baseline (speedup 1.0000x reference)
import functools

import jax
import jax.numpy as jnp
from jax import lax
from jax.experimental import pallas as pl
from jax.experimental.pallas import tpu as pltpu
from jax.experimental.pallas import tpu_sc as plsc

VOCAB = 100000
VOCAB_P = 100352
EMBED = 16
WINDOW = 20
BATCH = 1024

NUM_CORES = 2
NUM_SUBCORES = 16
NW = NUM_CORES * NUM_SUBCORES
B_PER_W = BATCH // NW
ROWS_PER_W = B_PER_W * WINDOW

VBLK = 4096
VGRID = (VOCAB + VBLK - 1) // VBLK


def _prep_body(tt_ref, o_ref):
    t = tt_ref[...]
    ssq = jnp.sum(t * t, axis=0, keepdims=True)
    norm = jnp.sqrt(ssq)
    scale = jnp.where(ssq > jnp.float32(1.0),
                      jnp.float32(1.0) / (norm + jnp.float32(1e-7)),
                      jnp.float32(1.0))
    o_ref[:, :VOCAB] = t * scale


def _tc_prep(table_t):
    return pl.pallas_call(
        _prep_body,
        out_shape=jax.ShapeDtypeStruct((EMBED, VOCAB_P), jnp.float32),
    )(table_t)


def _sc_pool_body(idx_hbm, table_hbm, out_hbm, idx_v, idxw_v, rows_v,
                  pool_v, sem0, sem1):
    wid = lax.axis_index("s") * NUM_CORES + lax.axis_index("c")

    for w in range(WINDOW):
        src = idx_hbm.at[pl.ds(w * BATCH + wid * B_PER_W, B_PER_W)]
        pltpu.async_copy(src, idx_v.at[pl.ds(w * B_PER_W, B_PER_W)], sem0)
    for w in range(WINDOW):
        pltpu.make_async_copy(
            idx_hbm.at[pl.ds(w * BATCH + wid * B_PER_W, B_PER_W)],
            idx_v.at[pl.ds(w * B_PER_W, B_PER_W)], sem0).wait()

    def mkidx(c, carry):
        base = idx_v[pl.ds(c * 16, 16)]
        def per_e(e, carry2):
            idxw_v[pl.ds(e * ROWS_PER_W + c * 16, 16)] = base + e * VOCAB_P
            return carry2
        return lax.fori_loop(0, EMBED, per_e, carry)

    lax.fori_loop(0, ROWS_PER_W // 16, mkidx, 0)

    for e in range(EMBED):
        pltpu.async_copy(
            table_hbm.at[idxw_v.at[pl.ds(e * ROWS_PER_W, ROWS_PER_W)]],
            rows_v.at[pl.ds(e * ROWS_PER_W, ROWS_PER_W)], sem1)
    for e in range(EMBED):
        pltpu.make_async_copy(
            table_hbm.at[idxw_v.at[pl.ds(e * ROWS_PER_W, ROWS_PER_W)]],
            rows_v.at[pl.ds(e * ROWS_PER_W, ROWS_PER_W)], sem1).wait()

    def outer(i, carry):
        e = i // 2
        bc = (i % 2) * 16

        def inner(w, acc):
            return acc + rows_v[pl.ds(e * ROWS_PER_W + w * B_PER_W + bc, 16)]

        acc = lax.fori_loop(0, WINDOW, inner, jnp.zeros((16,), jnp.float32))
        pool_v[e, pl.ds(bc, 16)] = acc
        return carry

    lax.fori_loop(0, EMBED * 2, outer, 0)
    pltpu.sync_copy(pool_v, out_hbm.at[pl.ds(wid * EMBED, EMBED), :])


@functools.cache
def _sc_pool():
    return pl.kernel(
        _sc_pool_body,
        mesh=plsc.VectorSubcoreMesh(core_axis_name="c", subcore_axis_name="s"),
        out_type=jax.ShapeDtypeStruct((NW * EMBED, B_PER_W), jnp.float32),
        scratch_types=[
            pltpu.VMEM((ROWS_PER_W,), jnp.int32),
            pltpu.VMEM((EMBED * ROWS_PER_W,), jnp.int32),
            pltpu.VMEM((EMBED * ROWS_PER_W,), jnp.float32),
            pltpu.VMEM((EMBED, B_PER_W), jnp.float32),
            pltpu.SemaphoreType.DMA,
            pltpu.SemaphoreType.DMA,
        ],
        compiler_params=pltpu.CompilerParams(needs_layout_passes=False),
    )


def _mm_body(wt_ref, pt_ref, b_ref, o_ref):
    wk = jnp.concatenate([wt_ref[...], b_ref[...]], axis=0)
    p17 = jnp.concatenate(
        [pt_ref[...] * jnp.float32(1.0 / WINDOW),
         jnp.ones((1, BATCH), jnp.float32)], axis=0)
    o_ref[...] = lax.dot_general(
        wk, p17,
        dimension_numbers=(((0,), (0,)), ((), ())),
        preferred_element_type=jnp.float32,
    )


def _tc_project(W_t, pooled_t, b2):
    return pl.pallas_call(
        _mm_body,
        grid=(VGRID,),
        in_specs=[
            pl.BlockSpec((EMBED, VBLK), lambda j: (0, j)),
            pl.BlockSpec((EMBED, BATCH), lambda j: (0, 0)),
            pl.BlockSpec((1, VBLK), lambda j: (0, j)),
        ],
        out_specs=pl.BlockSpec((VBLK, BATCH), lambda j: (j, 0)),
        out_shape=jax.ShapeDtypeStruct((VOCAB, BATCH), jnp.float32),
        compiler_params=pltpu.CompilerParams(
            dimension_semantics=("parallel",),
            fuse_transposed_lhs_in_matmul=True,
        ),
    )(W_t, pooled_t, b2)


def kernel(x, table, W, b):
    idx_t = x.T.reshape(-1).astype(jnp.int32)
    table_s = _tc_prep(table.T)
    pooled_blk = _sc_pool()(idx_t, table_s.reshape(-1))
    pooled_t = (pooled_blk.reshape(NW, EMBED, B_PER_W)
                .transpose(1, 0, 2).reshape(EMBED, BATCH))
    out_t = _tc_project(W.T, pooled_t, b.reshape(1, VOCAB))
    return out_t.T


# speedup vs baseline: 1.0076x; 1.0076x over previous
import functools

import jax
import jax.numpy as jnp
from jax import lax
from jax.experimental import pallas as pl
from jax.experimental.pallas import tpu as pltpu
from jax.experimental.pallas import tpu_sc as plsc

VOCAB = 100000
EMBED = 16
WINDOW = 20
BATCH = 1024
LANES = 128

NUM_CORES = 2
NUM_SUBCORES = 16
NW = NUM_CORES * NUM_SUBCORES
B_PER_W = BATCH // NW

WBLK = 4096
WGRID = (VOCAB + WBLK - 1) // WBLK
VBLK = 2048
VGRID = (VOCAB + VBLK - 1) // VBLK


def _prep_body(tt_ref, o_ref):
    t = tt_ref[...]
    ssq = jnp.sum(t * t, axis=0, keepdims=True)
    norm = jnp.sqrt(ssq)
    scale = jnp.where(ssq > jnp.float32(1.0),
                      jnp.float32(1.0) / (norm + jnp.float32(1e-7)),
                      jnp.float32(1.0))
    o_ref[:, :EMBED] = (t * scale).T


def _tc_prep(table_t):
    return pl.pallas_call(
        _prep_body,
        grid=(WGRID,),
        in_specs=[pl.BlockSpec((EMBED, WBLK), lambda j: (0, j))],
        out_specs=pl.BlockSpec((WBLK, LANES), lambda j: (j, 0)),
        out_shape=jax.ShapeDtypeStruct((VOCAB, LANES), jnp.float32),
        compiler_params=pltpu.CompilerParams(
            dimension_semantics=("parallel",),
        ),
    )(table_t)


def _sc_pool_body(idx_hbm, table_hbm, out_hbm, idx_v, rows_v, pool_v,
                  sem0, sem1):
    wid = lax.axis_index("s") * NUM_CORES + lax.axis_index("c")

    for w in range(WINDOW):
        pltpu.async_copy(idx_hbm.at[pl.ds(w * BATCH + wid * B_PER_W, B_PER_W)],
                         idx_v.at[pl.ds(w * B_PER_W, B_PER_W)], sem0)
    for w in range(WINDOW):
        pltpu.make_async_copy(
            idx_hbm.at[pl.ds(w * BATCH + wid * B_PER_W, B_PER_W)],
            idx_v.at[pl.ds(w * B_PER_W, B_PER_W)], sem0).wait()

    for c in range(WINDOW * B_PER_W // 128):
        pltpu.async_copy(table_hbm.at[idx_v.at[pl.ds(c * 128, 128)]],
                         rows_v.at[pl.ds(c * 128, 128)], sem1)
    for c in range(WINDOW * B_PER_W // 128):
        pltpu.make_async_copy(table_hbm.at[idx_v.at[pl.ds(c * 128, 128)]],
                              rows_v.at[pl.ds(c * 128, 128)], sem1).wait()

    def outer(b, carry):
        def inner(w, acc):
            return acc + rows_v[w * B_PER_W + b, pl.ds(0, 16)]

        acc = lax.fori_loop(0, WINDOW, inner, jnp.zeros((16,), jnp.float32))
        pool_v[b, pl.ds(0, 16)] = acc
        return carry

    lax.fori_loop(0, B_PER_W, outer, 0)
    pltpu.sync_copy(pool_v, out_hbm.at[pl.ds(wid * B_PER_W, B_PER_W)])


@functools.cache
def _sc_pool():
    return pl.kernel(
        _sc_pool_body,
        mesh=plsc.VectorSubcoreMesh(core_axis_name="c", subcore_axis_name="s"),
        out_type=jax.ShapeDtypeStruct((BATCH, LANES), jnp.float32),
        scratch_types=[
            pltpu.VMEM((B_PER_W * WINDOW,), jnp.int32),
            pltpu.VMEM((B_PER_W * WINDOW, LANES), jnp.float32),
            pltpu.VMEM((B_PER_W, LANES), jnp.float32),
            pltpu.SemaphoreType.DMA,
            pltpu.SemaphoreType.DMA,
        ],
        compiler_params=pltpu.CompilerParams(needs_layout_passes=False),
    )


def _mm_body(wt_ref, p_ref, b_ref, o_ref):
    wk = jnp.concatenate([wt_ref[...], b_ref[...]], axis=0)
    p17 = jnp.concatenate(
        [p_ref[:, :EMBED] * jnp.float32(1.0 / WINDOW),
         jnp.ones((BATCH, 1), jnp.float32)], axis=1)
    o_ref[...] = lax.dot_general(
        wk, p17,
        dimension_numbers=(((0,), (1,)), ((), ())),
        preferred_element_type=jnp.float32,
    )


def _tc_project(W_t, pooled, b2):
    return pl.pallas_call(
        _mm_body,
        grid=(VGRID,),
        in_specs=[
            pl.BlockSpec((EMBED, VBLK), lambda j: (0, j)),
            pl.BlockSpec((BATCH, LANES), lambda j: (0, 0)),
            pl.BlockSpec((1, VBLK), lambda j: (0, j)),
        ],
        out_specs=pl.BlockSpec((VBLK, BATCH), lambda j: (j, 0)),
        out_shape=jax.ShapeDtypeStruct((VOCAB, BATCH), jnp.float32),
        compiler_params=pltpu.CompilerParams(
            dimension_semantics=("parallel",),
            fuse_transposed_lhs_in_matmul=True,
        ),
    )(W_t, pooled, b2)


def kernel(x, table, W, b):
    idx_t = x.T.reshape(-1).astype(jnp.int32)
    table_sw = _tc_prep(table.T)
    pooled = _sc_pool()(idx_t, table_sw)
    out_t = _tc_project(W.T, pooled, b.reshape(1, VOCAB))
    return out_t.T
